# baseline (speedup 1.0000x reference)
import functools

import jax
import jax.numpy as jnp
from jax import lax
from jax.experimental import pallas as pl
from jax.experimental.pallas import tpu as pltpu
from jax.experimental.pallas import tpu_sc as plsc

NC = 2
NS = 16
NW = NC * NS
CHUNK = 128
NBUF = 8


def _build_gather(total: int, emb_dim: int):
    n_chunks = total // (NW * CHUNK)
    assert n_chunks * NW * CHUNK == total
    assert n_chunks % NBUF == 0
    rows_per_worker = n_chunks * CHUNK

    mesh = plsc.VectorSubcoreMesh(
        core_axis_name="c", subcore_axis_name="s",
        num_cores=NC, num_subcores=NS)

    @functools.partial(
        pl.kernel,
        out_type=jax.ShapeDtypeStruct((total, emb_dim), jnp.float32),
        mesh=mesh,
        scratch_types=[
            pltpu.VMEM((n_chunks, CHUNK), jnp.int32),
            pltpu.VMEM((NBUF, CHUNK, emb_dim), jnp.float32),
            pltpu.SemaphoreType.DMA,
            pltpu.SemaphoreType.DMA,
        ],
        compiler_params=pltpu.CompilerParams(use_tc_tiling_on_sc=False),
    )
    def emb_gather(table_hbm, idx_hbm, out_hbm, idx_v, bufs, sem_g, sem_o):
        wid = lax.axis_index("s") * NC + lax.axis_index("c")
        base = wid * rows_per_worker
        pltpu.sync_copy(idx_hbm.at[wid], idx_v)

        def outer(i, carry):
            g0 = i * NBUF
            gathers = []
            for b in range(NBUF):
                gathers.append(pltpu.async_copy(
                    table_hbm.at[idx_v.at[g0 + b]], bufs.at[b], sem_g))
            outs = []
            for b in range(NBUF):
                gathers[b].wait()
                outs.append(pltpu.async_copy(
                    bufs.at[b],
                    out_hbm.at[pl.ds(base + (g0 + b) * CHUNK, CHUNK)],
                    sem_o))
            for o in outs:
                o.wait()
            return carry

        lax.fori_loop(0, n_chunks // NBUF, outer, 0)

    return emb_gather


def kernel(batch_input, lengths, embedding_table):
    del lengths
    batch, hist = batch_input.shape
    vocab, emb_dim = embedding_table.shape
    total = batch * hist
    n_chunks = total // (NW * CHUNK)
    idx = batch_input.reshape(NW, n_chunks, CHUNK).astype(jnp.int32)
    out = _build_gather(total, emb_dim)(embedding_table, idx)
    return out.reshape(batch, hist, emb_dim)


# speedup vs baseline: 1.0014x; 1.0014x over previous
import functools

import jax
import jax.numpy as jnp
from jax import lax
from jax.experimental import pallas as pl
from jax.experimental.pallas import tpu as pltpu
from jax.experimental.pallas import tpu_sc as plsc

NC = 2
NS = 16
NW = NC * NS
CHUNK = 128
NBUF = 10


def _build_gather(total: int, emb_dim: int):
    n_chunks = total // (NW * CHUNK)
    assert n_chunks * NW * CHUNK == total
    assert n_chunks % NBUF == 0
    rows_per_worker = n_chunks * CHUNK

    mesh = plsc.VectorSubcoreMesh(
        core_axis_name="c", subcore_axis_name="s",
        num_cores=NC, num_subcores=NS)

    @functools.partial(
        pl.kernel,
        out_type=jax.ShapeDtypeStruct((total, emb_dim), jnp.float32),
        mesh=mesh,
        scratch_types=[
            pltpu.VMEM((n_chunks, CHUNK), jnp.int32),
            pltpu.VMEM((NBUF, CHUNK, emb_dim), jnp.float32),
            pltpu.SemaphoreType.DMA,
            pltpu.SemaphoreType.DMA,
        ],
        compiler_params=pltpu.CompilerParams(use_tc_tiling_on_sc=False),
    )
    def emb_gather(table_hbm, idx_hbm, out_hbm, idx_v, bufs, sem_g, sem_o):
        wid = lax.axis_index("s") * NC + lax.axis_index("c")
        base = wid * rows_per_worker
        pltpu.sync_copy(idx_hbm.at[wid], idx_v)

        def gather_desc(g, b):
            return pltpu.make_async_copy(table_hbm.at[idx_v.at[g]],
                                         bufs.at[b], sem_g)

        def scatter_desc(g, b):
            return pltpu.make_async_copy(
                bufs.at[b], out_hbm.at[pl.ds(base + g * CHUNK, CHUNK)], sem_o)

        def gather(g, b):
            gather_desc(g, b).start()

        def scatter(g, b):
            scatter_desc(g, b).start()

        for b in range(NBUF):
            gather(b, b)

        def outer(i, carry):
            g0 = i * NBUF
            for b in range(NBUF):
                g = g0 + b
                gather_desc(g, b).wait()
                scatter(g, b)

                @pl.when(i < n_chunks // NBUF - 1)
                def _():
                    scatter_desc(g, b).wait()
                    gather(g + NBUF, b)

            return carry

        lax.fori_loop(0, n_chunks // NBUF, outer, 0)

        for b in range(NBUF):
            scatter_desc(n_chunks - NBUF + b, b).wait()

    return emb_gather


def kernel(batch_input, lengths, embedding_table):
    del lengths
    batch, hist = batch_input.shape
    vocab, emb_dim = embedding_table.shape
    total = batch * hist
    n_chunks = total // (NW * CHUNK)
    idx = batch_input.reshape(NW, n_chunks, CHUNK).astype(jnp.int32)
    out = _build_gather(total, emb_dim)(embedding_table, idx)
    return out.reshape(batch, hist, emb_dim)


# speedup vs baseline: 1.2200x; 1.2183x over previous
import functools

import jax
import jax.numpy as jnp
from jax import lax
from jax.experimental import pallas as pl
from jax.experimental.pallas import tpu as pltpu
from jax.experimental.pallas import tpu_sc as plsc

NC = 2
NS = 16
NW = NC * NS
CHUNK = 128
NBUF = 5
PRIME = 4
LAG = NBUF - PRIME
PAD = 128


def _build_gather(total: int, emb_dim: int):
    n_chunks = total // (NW * CHUNK)
    assert n_chunks * NW * CHUNK == total
    assert n_chunks % NBUF == 0 and n_chunks >= 2 * NBUF
    n_outer = n_chunks // NBUF
    rows_per_worker = n_chunks * CHUNK

    mesh = plsc.VectorSubcoreMesh(
        core_axis_name="c", subcore_axis_name="s",
        num_cores=NC, num_subcores=NS)

    @functools.partial(
        pl.kernel,
        out_type=jax.ShapeDtypeStruct((total, PAD), jnp.float32),
        mesh=mesh,
        scratch_types=[
            pltpu.VMEM((n_chunks, CHUNK), jnp.int32),
            pltpu.VMEM((NBUF, CHUNK, PAD), jnp.float32),
            pltpu.SemaphoreType.DMA,
            pltpu.SemaphoreType.DMA,
        ],
        compiler_params=pltpu.CompilerParams(use_tc_tiling_on_sc=True),
    )
    def emb_gather(table_hbm, idx_hbm, out_hbm, idx_v, bufs, sem_g, sem_o):
        wid = lax.axis_index("s") * NC + lax.axis_index("c")
        base = wid * rows_per_worker
        pltpu.sync_copy(idx_hbm.at[wid], idx_v)

        def gather_desc(g, b):
            return pltpu.make_async_copy(table_hbm.at[idx_v.at[g]],
                                         bufs.at[b], sem_g)

        def scatter_desc(g, b):
            return pltpu.make_async_copy(
                bufs.at[b],
                out_hbm.at[pl.ds(base + g * CHUNK, CHUNK)], sem_o)

        for b in range(PRIME):
            gather_desc(b, b).start()

        def step(t, b, first, last):
            gather_desc(t, b).wait()
            scatter_desc(t, b).start()
            if first and b < LAG:
                gather_desc(t + PRIME, (b + PRIME) % NBUF).start()
            elif not (last and b >= LAG):
                scatter_desc(t - LAG, (b - LAG) % NBUF).wait()
                gather_desc(t + PRIME, (b + PRIME) % NBUF).start()
            else:
                scatter_desc(t - LAG, (b - LAG) % NBUF).wait()

        for b in range(NBUF):
            step(b, b, True, False)

        def outer(i, carry):
            t0 = i * NBUF
            for b in range(NBUF):
                step(t0 + b, b, False, False)
            return carry

        lax.fori_loop(1, n_outer - 1, outer, 0)

        for b in range(NBUF):
            step(n_chunks - NBUF + b, b, False, True)

        for k in range(LAG):
            g = n_chunks - LAG + k
            scatter_desc(g, g % NBUF).wait()

    return emb_gather


def kernel(batch_input, lengths, embedding_table):
    del lengths
    batch, hist = batch_input.shape
    vocab, emb_dim = embedding_table.shape
    total = batch * hist
    n_chunks = total // (NW * CHUNK)
    idx = batch_input.reshape(NW, n_chunks, CHUNK).astype(jnp.int32)
    table_padded = jnp.pad(embedding_table, ((0, 0), (0, PAD - emb_dim)))
    out = _build_gather(total, emb_dim)(table_padded, idx)
    return out[:, :emb_dim].reshape(batch, hist, emb_dim)
